# baseline (speedup 1.0000x reference)
import jax
import jax.numpy as jnp
from jax import lax
from jax.experimental import pallas as pl
from jax.experimental.pallas import tpu as pltpu
from jax.experimental.pallas import tpu_sc as plsc

N = 10000
D = 128
E = 320000

NC = 2
NS = 16
NW = NC * NS
K = 128
CH = 80
EPW = K * CH
EPAD = EPW * NW
NPAD = 10240
ZROWS = NPAD // NS
OROWS = 624
DW = 16
NBUF = 2
LAG = 4

_mesh = plsc.VectorSubcoreMesh(core_axis_name="c", subcore_axis_name="s")


def _deg_body(dst_hbm, out_hbm, dst_all, buf_v, acc_sh, sem0, sem1, sem2,
              sem3, zsem):
    sem = (sem0, sem1, sem2, sem3)
    c = lax.axis_index("c")
    s = lax.axis_index("s")
    wid = s * NC + c

    pltpu.sync_copy(dst_hbm.at[wid], dst_all)

    def _zrow(i, carry):
        for j in range(D // 16):
            buf_v[i, pl.ds(j * 16, 16)] = jnp.zeros((16,), jnp.float32)
        return carry

    lax.fori_loop(0, K, _zrow, 0)
    for m in range(ZROWS // K):
        pltpu.async_copy(buf_v, acc_sh.at[pl.ds(s * ZROWS + m * K, K)], zsem)
    for m in range(ZROWS // K):
        pltpu.make_async_copy(buf_v, acc_sh.at[pl.ds(s * ZROWS + m * K, K)],
                              zsem).wait()

    def _orow(i, carry):
        for j in range(D // 16):
            buf_v[i, pl.ds(j * 16, 16)] = jnp.ones((16,), jnp.float32)
        return carry

    lax.fori_loop(0, K, _orow, 0)
    plsc.subcore_barrier()

    for b in range(LAG):
        pltpu.async_copy(buf_v, acc_sh.at[dst_all.at[b]], sem[b], add=True)

    def _fire(g, carry):
        for b in range(LAG):
            j = g * LAG + b
            pltpu.make_async_copy(buf_v, acc_sh.at[dst_all.at[j - LAG]],
                                  sem[b]).wait()
            pltpu.async_copy(buf_v, acc_sh.at[dst_all.at[j]], sem[b],
                             add=True)
        return carry

    lax.fori_loop(1, CH // LAG, _fire, 0)
    for b in range(LAG):
        pltpu.make_async_copy(buf_v, acc_sh.at[dst_all.at[CH - LAG + b]],
                              sem[b]).wait()
    plsc.subcore_barrier()

    pltpu.sync_copy(acc_sh.at[pl.ds(s * OROWS, OROWS)],
                    out_hbm.at[c, pl.ds(s * OROWS, OROWS)])

    @pl.when(s == NS - 1)
    def _tail():
        pltpu.sync_copy(acc_sh.at[pl.ds(NS * OROWS, N - NS * OROWS)],
                        out_hbm.at[c, pl.ds(NS * OROWS, N - NS * OROWS)])


_deg_kernel = pl.kernel(
    _deg_body,
    out_type=jax.ShapeDtypeStruct((NC, N, D), jnp.float32),
    mesh=_mesh,
    scratch_types=[
        pltpu.VMEM((CH, K), jnp.int32),
        pltpu.VMEM((K, D), jnp.float32),
        pltpu.VMEM_SHARED((NPAD, D), jnp.float32),
    ] + [pltpu.SemaphoreType.DMA] * 5,
)


def _edge_body(g_hbm, src_hbm, dst_hbm, out_hbm, sbuf, dst_all, rows_v,
               acc_sh, isem0, isem1, gsem0, gsem1, ssem0, ssem1, zsem):
    isem = (isem0, isem1)
    gsem = (gsem0, gsem1)
    ssem = (ssem0, ssem1)
    c = lax.axis_index("c")
    s = lax.axis_index("s")
    wid = s * NC + c

    pltpu.sync_copy(dst_hbm.at[wid], dst_all)

    def _zrow(i, carry):
        for j in range(D // 16):
            rows_v[0, i, pl.ds(j * 16, 16)] = jnp.zeros((16,), jnp.float32)
        return carry

    lax.fori_loop(0, K, _zrow, 0)
    for m in range(ZROWS // K):
        pltpu.async_copy(rows_v.at[0], acc_sh.at[pl.ds(s * ZROWS + m * K, K)],
                         zsem)
    for m in range(ZROWS // K):
        pltpu.make_async_copy(rows_v.at[0],
                              acc_sh.at[pl.ds(s * ZROWS + m * K, K)],
                              zsem).wait()
    plsc.subcore_barrier()

    ebase = wid * EPW
    for b in range(NBUF):
        pltpu.async_copy(src_hbm.at[pl.ds(ebase + b * K, K)], sbuf.at[b],
                         isem[b])
    for b in range(NBUF):
        pltpu.make_async_copy(src_hbm.at[pl.ds(ebase + b * K, K)], sbuf.at[b],
                              isem[b]).wait()
        pltpu.async_copy(g_hbm.at[sbuf.at[b]], rows_v.at[b], gsem[b])

    def _cycle(g, carry):
        for b in range(NBUF):
            j = g * NBUF + b
            pltpu.make_async_copy(g_hbm.at[sbuf.at[b]], rows_v.at[b],
                                  gsem[b]).wait()
            pltpu.async_copy(rows_v.at[b], acc_sh.at[dst_all.at[j]], ssem[b],
                             add=True)

            @pl.when(j + NBUF < CH)
            def _(b=b, j=j):
                pltpu.async_copy(
                    src_hbm.at[pl.ds(ebase + (j + NBUF) * K, K)], sbuf.at[b],
                    isem[b])

        for b in range(NBUF):
            jn = (g + 1) * NBUF + b

            @pl.when(jn < CH)
            def _(b=b, jn=jn):
                pltpu.make_async_copy(rows_v.at[b],
                                      acc_sh.at[dst_all.at[jn - NBUF]],
                                      ssem[b]).wait()
                pltpu.make_async_copy(
                    src_hbm.at[pl.ds(ebase + jn * K, K)], sbuf.at[b],
                    isem[b]).wait()
                pltpu.async_copy(g_hbm.at[sbuf.at[b]], rows_v.at[b], gsem[b])

        return carry

    lax.fori_loop(0, CH // NBUF, _cycle, 0)
    for b in range(NBUF):
        pltpu.make_async_copy(rows_v.at[b],
                              acc_sh.at[dst_all.at[CH - NBUF + b]],
                              ssem[b]).wait()
    plsc.subcore_barrier()

    pltpu.sync_copy(acc_sh.at[pl.ds(s * OROWS, OROWS)],
                    out_hbm.at[c, pl.ds(s * OROWS, OROWS)])

    @pl.when(s == NS - 1)
    def _tail():
        pltpu.sync_copy(acc_sh.at[pl.ds(NS * OROWS, N - NS * OROWS)],
                        out_hbm.at[c, pl.ds(NS * OROWS, N - NS * OROWS)])


_edge_kernel = pl.kernel(
    _edge_body,
    out_type=jax.ShapeDtypeStruct((NC, N, D), jnp.float32),
    mesh=_mesh,
    scratch_types=[
        pltpu.VMEM((NBUF, K), jnp.int32),
        pltpu.VMEM((CH, K), jnp.int32),
        pltpu.VMEM((NBUF, K, D), jnp.float32),
        pltpu.VMEM_SHARED((NPAD, D), jnp.float32),
    ] + [pltpu.SemaphoreType.DMA] * (3 * NBUF + 1),
)


R = 1000


def _tc1_body(x_ref, w1_ref, p_ref, h_ref, g_ref, dinv_ref):
    p = p_ref[...]
    deg = 1.0 + p[0, :, 0:1] + p[1, :, 0:1]
    dinv = lax.rsqrt(deg)
    h = jnp.dot(x_ref[...], w1_ref[...], preferred_element_type=jnp.float32)
    h_ref[...] = h
    g_ref[...] = h * dinv
    dinv_ref[...] = dinv


def _tc1(x, W1, p):
    return pl.pallas_call(
        _tc1_body,
        grid=(N // R,),
        in_specs=[
            pl.BlockSpec((R, D), lambda i: (i, 0)),
            pl.BlockSpec((D, D), lambda i: (0, 0)),
            pl.BlockSpec((NC, R, D), lambda i: (0, i, 0)),
        ],
        out_specs=[
            pl.BlockSpec((R, D), lambda i: (i, 0)),
            pl.BlockSpec((R, D), lambda i: (i, 0)),
            pl.BlockSpec((R, 1), lambda i: (i, 0)),
        ],
        out_shape=[
            jax.ShapeDtypeStruct((N, D), jnp.float32),
            jax.ShapeDtypeStruct((N, D), jnp.float32),
            jax.ShapeDtypeStruct((N, 1), jnp.float32),
        ],
    )(x, W1, p)


def _tc2_body(acc_ref, h1_ref, dinv_ref, b1_ref, w2_ref, h2_ref, g2_ref):
    dinv = dinv_ref[...]
    a = acc_ref[0] + acc_ref[1]
    z = jnp.maximum(a * dinv + h1_ref[...] * (dinv * dinv) + b1_ref[...], 0.0)
    h2 = jnp.dot(z, w2_ref[...], preferred_element_type=jnp.float32)
    h2_ref[...] = h2
    g2_ref[...] = h2 * dinv


def _tc2(acc, h1, dinv, b1, W2):
    return pl.pallas_call(
        _tc2_body,
        grid=(N // R,),
        in_specs=[
            pl.BlockSpec((NC, R, D), lambda i: (0, i, 0)),
            pl.BlockSpec((R, D), lambda i: (i, 0)),
            pl.BlockSpec((R, 1), lambda i: (i, 0)),
            pl.BlockSpec((1, D), lambda i: (0, 0)),
            pl.BlockSpec((D, D), lambda i: (0, 0)),
        ],
        out_specs=[
            pl.BlockSpec((R, D), lambda i: (i, 0)),
            pl.BlockSpec((R, D), lambda i: (i, 0)),
        ],
        out_shape=[
            jax.ShapeDtypeStruct((N, D), jnp.float32),
            jax.ShapeDtypeStruct((N, D), jnp.float32),
        ],
    )(acc, h1, dinv, b1, W2)


def _tc3_body(acc_ref, h2_ref, dinv_ref, b2_ref, fw1_ref, fb1_ref, fw2_ref,
              fb2_ref, out_ref):
    dinv = dinv_ref[...]
    a = acc_ref[0] + acc_ref[1]
    z = jnp.maximum(a * dinv + h2_ref[...] * (dinv * dinv) + b2_ref[...], 0.0)
    t = jnp.maximum(
        jnp.dot(z, fw1_ref[...], preferred_element_type=jnp.float32)
        + fb1_ref[...], 0.0)
    out_ref[...] = (
        jnp.dot(t, fw2_ref[...], preferred_element_type=jnp.float32)
        + fb2_ref[...])


def _tc3(acc, h2, dinv, b2, fcW1, fcb1, fcW2, fcb2):
    return pl.pallas_call(
        _tc3_body,
        grid=(N // R,),
        in_specs=[
            pl.BlockSpec((NC, R, D), lambda i: (0, i, 0)),
            pl.BlockSpec((R, D), lambda i: (i, 0)),
            pl.BlockSpec((R, 1), lambda i: (i, 0)),
            pl.BlockSpec((1, D), lambda i: (0, 0)),
            pl.BlockSpec((D, D), lambda i: (0, 0)),
            pl.BlockSpec((1, D), lambda i: (0, 0)),
            pl.BlockSpec((D, D), lambda i: (0, 0)),
            pl.BlockSpec((1, D), lambda i: (0, 0)),
        ],
        out_specs=pl.BlockSpec((R, D), lambda i: (i, 0)),
        out_shape=jax.ShapeDtypeStruct((N, D), jnp.float32),
    )(acc, h2, dinv, b2, fcW1, fcb1, fcW2, fcb2)


def kernel(x, edge_index, W1, b1, W2, b2, fcW1, fcb1, fcW2, fcb2):
    src = edge_index[0]
    dst = edge_index[1]
    pad = EPAD - E
    src_p = jnp.concatenate([src, jnp.zeros((pad,), jnp.int32)])
    dst_p = jnp.concatenate([dst, jnp.full((pad,), N, jnp.int32)])
    dst_p = dst_p.reshape(NW, CH, K)

    p = _deg_kernel(dst_p)
    h1, g1, dinv = _tc1(x, W1, p)
    acc1 = _edge_kernel(g1, src_p, dst_p)
    h2, g2 = _tc2(acc1, h1, dinv, b1.reshape(1, D), W2)
    acc2 = _edge_kernel(g2, src_p, dst_p)
    return _tc3(acc2, h2, dinv, b2.reshape(1, D), fcW1, fcb1.reshape(1, D),
                fcW2, fcb2.reshape(1, D))


# speedup vs baseline: 1.0560x; 1.0560x over previous
import jax
import jax.numpy as jnp
from jax import lax
from jax.experimental import pallas as pl
from jax.experimental.pallas import tpu as pltpu
from jax.experimental.pallas import tpu_sc as plsc

N = 10000
D = 128
E = 320000

NC = 2
NS = 16
NW = NC * NS
K = 128
CH = 80
EPW = K * CH
EPAD = EPW * NW
CH0 = 112
CH1 = 48
EPP = 2 * EPW
NPAD = 10240
ZROWS = NPAD // NS
OROWS = 624
DW = 16
NBUF = 2
LAG = 4

_mesh = plsc.VectorSubcoreMesh(core_axis_name="c", subcore_axis_name="s")


def _deg_body(dst_hbm, out_hbm, dst_all, buf_v, acc_sh, sem0, sem1, sem2,
              sem3, zsem):
    sem = (sem0, sem1, sem2, sem3)
    c = lax.axis_index("c")
    s = lax.axis_index("s")
    wid = s * NC + c

    pltpu.sync_copy(dst_hbm.at[wid], dst_all)

    def _zrow(i, carry):
        for j in range(D // 16):
            buf_v[i, pl.ds(j * 16, 16)] = jnp.zeros((16,), jnp.float32)
        return carry

    lax.fori_loop(0, K, _zrow, 0)
    for m in range(ZROWS // K):
        pltpu.async_copy(buf_v, acc_sh.at[pl.ds(s * ZROWS + m * K, K)], zsem)
    for m in range(ZROWS // K):
        pltpu.make_async_copy(buf_v, acc_sh.at[pl.ds(s * ZROWS + m * K, K)],
                              zsem).wait()

    def _orow(i, carry):
        for j in range(D // 16):
            buf_v[i, pl.ds(j * 16, 16)] = jnp.ones((16,), jnp.float32)
        return carry

    lax.fori_loop(0, K, _orow, 0)
    plsc.subcore_barrier()

    for b in range(LAG):
        pltpu.async_copy(buf_v, acc_sh.at[dst_all.at[b]], sem[b], add=True)

    def _fire(g, carry):
        for b in range(LAG):
            j = g * LAG + b
            pltpu.make_async_copy(buf_v, acc_sh.at[dst_all.at[j - LAG]],
                                  sem[b]).wait()
            pltpu.async_copy(buf_v, acc_sh.at[dst_all.at[j]], sem[b],
                             add=True)
        return carry

    lax.fori_loop(1, CH // LAG, _fire, 0)
    for b in range(LAG):
        pltpu.make_async_copy(buf_v, acc_sh.at[dst_all.at[CH - LAG + b]],
                              sem[b]).wait()
    plsc.subcore_barrier()

    pltpu.sync_copy(acc_sh.at[pl.ds(s * OROWS, OROWS)],
                    out_hbm.at[c, pl.ds(s * OROWS, OROWS)])

    @pl.when(s == NS - 1)
    def _tail():
        pltpu.sync_copy(acc_sh.at[pl.ds(NS * OROWS, N - NS * OROWS)],
                        out_hbm.at[c, pl.ds(NS * OROWS, N - NS * OROWS)])


_deg_kernel = pl.kernel(
    _deg_body,
    out_type=jax.ShapeDtypeStruct((NC, N, D), jnp.float32),
    mesh=_mesh,
    scratch_types=[
        pltpu.VMEM((CH, K), jnp.int32),
        pltpu.VMEM((K, D), jnp.float32),
        pltpu.VMEM_SHARED((NPAD, D), jnp.float32),
    ] + [pltpu.SemaphoreType.DMA] * 5,
)


def _edge_body(g_hbm, src_hbm, dst0_hbm, dst1_hbm, out_hbm, sbuf, dst_all,
               rows_v, acc_sh, isem0, isem1, gsem0, gsem1, ssem0, ssem1,
               zsem):
    isem = (isem0, isem1)
    gsem = (gsem0, gsem1)
    ssem = (ssem0, ssem1)
    c = lax.axis_index("c")
    s = lax.axis_index("s")

    @pl.when(c == 0)
    def _stage0():
        pltpu.sync_copy(dst0_hbm.at[s], dst_all.at[pl.ds(0, CH0)])

    @pl.when(c == 1)
    def _stage1():
        pltpu.sync_copy(dst1_hbm.at[s], dst_all.at[pl.ds(0, CH1)])

    def _zrow(i, carry):
        for j in range(D // 16):
            rows_v[0, i, pl.ds(j * 16, 16)] = jnp.zeros((16,), jnp.float32)
        return carry

    lax.fori_loop(0, K, _zrow, 0)
    for m in range(ZROWS // K):
        pltpu.async_copy(rows_v.at[0], acc_sh.at[pl.ds(s * ZROWS + m * K, K)],
                         zsem)
    for m in range(ZROWS // K):
        pltpu.make_async_copy(rows_v.at[0],
                              acc_sh.at[pl.ds(s * ZROWS + m * K, K)],
                              zsem).wait()
    plsc.subcore_barrier()

    @pl.when(c == 0)
    def _run0():
        _run_pipeline(g_hbm, src_hbm, sbuf, rows_v, acc_sh, isem, gsem, ssem,
                      dst_all, s * EPP, CH0)

    @pl.when(c == 1)
    def _run1():
        _run_pipeline(g_hbm, src_hbm, sbuf, rows_v, acc_sh, isem, gsem, ssem,
                      dst_all, s * EPP + CH0 * K, CH1)

    _edge_out(out_hbm, acc_sh, c, s)


def _run_pipeline(g_hbm, src_hbm, sbuf, rows_v, acc_sh, isem, gsem, ssem,
                  dst_all, ebase, nch):
    for b in range(NBUF):
        pltpu.async_copy(src_hbm.at[pl.ds(ebase + b * K, K)], sbuf.at[b],
                         isem[b])
    for b in range(NBUF):
        pltpu.make_async_copy(src_hbm.at[pl.ds(ebase + b * K, K)], sbuf.at[b],
                              isem[b]).wait()
        pltpu.async_copy(g_hbm.at[sbuf.at[b]], rows_v.at[b], gsem[b])

    def _cycle(g, carry):
        for b in range(NBUF):
            j = g * NBUF + b
            pltpu.make_async_copy(g_hbm.at[sbuf.at[b]], rows_v.at[b],
                                  gsem[b]).wait()
            pltpu.async_copy(rows_v.at[b], acc_sh.at[dst_all.at[j]], ssem[b],
                             add=True)

            @pl.when(j + NBUF < nch)
            def _(b=b, j=j):
                pltpu.async_copy(
                    src_hbm.at[pl.ds(ebase + (j + NBUF) * K, K)], sbuf.at[b],
                    isem[b])

        for b in range(NBUF):
            jn = (g + 1) * NBUF + b

            @pl.when(jn < nch)
            def _(b=b, jn=jn):
                pltpu.make_async_copy(rows_v.at[b],
                                      acc_sh.at[dst_all.at[jn - NBUF]],
                                      ssem[b]).wait()
                pltpu.make_async_copy(
                    src_hbm.at[pl.ds(ebase + jn * K, K)], sbuf.at[b],
                    isem[b]).wait()
                pltpu.async_copy(g_hbm.at[sbuf.at[b]], rows_v.at[b], gsem[b])

        return carry

    lax.fori_loop(0, nch // NBUF, _cycle, 0)
    for b in range(NBUF):
        pltpu.make_async_copy(rows_v.at[b],
                              acc_sh.at[dst_all.at[nch - NBUF + b]],
                              ssem[b]).wait()


def _edge_out(out_hbm, acc_sh, c, s):
    plsc.subcore_barrier()

    pltpu.sync_copy(acc_sh.at[pl.ds(s * OROWS, OROWS)],
                    out_hbm.at[c, pl.ds(s * OROWS, OROWS)])

    @pl.when(s == NS - 1)
    def _tail():
        pltpu.sync_copy(acc_sh.at[pl.ds(NS * OROWS, N - NS * OROWS)],
                        out_hbm.at[c, pl.ds(NS * OROWS, N - NS * OROWS)])


_edge_kernel = pl.kernel(
    _edge_body,
    out_type=jax.ShapeDtypeStruct((NC, N, D), jnp.float32),
    mesh=_mesh,
    scratch_types=[
        pltpu.VMEM((NBUF, K), jnp.int32),
        pltpu.VMEM((CH0, K), jnp.int32),
        pltpu.VMEM((NBUF, K, D), jnp.float32),
        pltpu.VMEM_SHARED((NPAD, D), jnp.float32),
    ] + [pltpu.SemaphoreType.DMA] * (3 * NBUF + 1),
)


R = 1000


def _tc1_body(x_ref, w1_ref, p_ref, h_ref, g_ref, dinv_ref):
    p = p_ref[...]
    deg = 1.0 + p[0, :, 0:1] + p[1, :, 0:1]
    dinv = lax.rsqrt(deg)
    h = jnp.dot(x_ref[...], w1_ref[...], preferred_element_type=jnp.float32)
    h_ref[...] = h
    g_ref[...] = h * dinv
    dinv_ref[...] = dinv


def _tc1(x, W1, p):
    return pl.pallas_call(
        _tc1_body,
        grid=(N // R,),
        in_specs=[
            pl.BlockSpec((R, D), lambda i: (i, 0)),
            pl.BlockSpec((D, D), lambda i: (0, 0)),
            pl.BlockSpec((NC, R, D), lambda i: (0, i, 0)),
        ],
        out_specs=[
            pl.BlockSpec((R, D), lambda i: (i, 0)),
            pl.BlockSpec((R, D), lambda i: (i, 0)),
            pl.BlockSpec((R, 1), lambda i: (i, 0)),
        ],
        out_shape=[
            jax.ShapeDtypeStruct((N, D), jnp.float32),
            jax.ShapeDtypeStruct((N, D), jnp.float32),
            jax.ShapeDtypeStruct((N, 1), jnp.float32),
        ],
    )(x, W1, p)


def _tc2_body(acc_ref, h1_ref, dinv_ref, b1_ref, w2_ref, h2_ref, g2_ref):
    dinv = dinv_ref[...]
    a = acc_ref[0] + acc_ref[1]
    z = jnp.maximum(a * dinv + h1_ref[...] * (dinv * dinv) + b1_ref[...], 0.0)
    h2 = jnp.dot(z, w2_ref[...], preferred_element_type=jnp.float32)
    h2_ref[...] = h2
    g2_ref[...] = h2 * dinv


def _tc2(acc, h1, dinv, b1, W2):
    return pl.pallas_call(
        _tc2_body,
        grid=(N // R,),
        in_specs=[
            pl.BlockSpec((NC, R, D), lambda i: (0, i, 0)),
            pl.BlockSpec((R, D), lambda i: (i, 0)),
            pl.BlockSpec((R, 1), lambda i: (i, 0)),
            pl.BlockSpec((1, D), lambda i: (0, 0)),
            pl.BlockSpec((D, D), lambda i: (0, 0)),
        ],
        out_specs=[
            pl.BlockSpec((R, D), lambda i: (i, 0)),
            pl.BlockSpec((R, D), lambda i: (i, 0)),
        ],
        out_shape=[
            jax.ShapeDtypeStruct((N, D), jnp.float32),
            jax.ShapeDtypeStruct((N, D), jnp.float32),
        ],
    )(acc, h1, dinv, b1, W2)


def _tc3_body(acc_ref, h2_ref, dinv_ref, b2_ref, fw1_ref, fb1_ref, fw2_ref,
              fb2_ref, out_ref):
    dinv = dinv_ref[...]
    a = acc_ref[0] + acc_ref[1]
    z = jnp.maximum(a * dinv + h2_ref[...] * (dinv * dinv) + b2_ref[...], 0.0)
    t = jnp.maximum(
        jnp.dot(z, fw1_ref[...], preferred_element_type=jnp.float32)
        + fb1_ref[...], 0.0)
    out_ref[...] = (
        jnp.dot(t, fw2_ref[...], preferred_element_type=jnp.float32)
        + fb2_ref[...])


def _tc3(acc, h2, dinv, b2, fcW1, fcb1, fcW2, fcb2):
    return pl.pallas_call(
        _tc3_body,
        grid=(N // R,),
        in_specs=[
            pl.BlockSpec((NC, R, D), lambda i: (0, i, 0)),
            pl.BlockSpec((R, D), lambda i: (i, 0)),
            pl.BlockSpec((R, 1), lambda i: (i, 0)),
            pl.BlockSpec((1, D), lambda i: (0, 0)),
            pl.BlockSpec((D, D), lambda i: (0, 0)),
            pl.BlockSpec((1, D), lambda i: (0, 0)),
            pl.BlockSpec((D, D), lambda i: (0, 0)),
            pl.BlockSpec((1, D), lambda i: (0, 0)),
        ],
        out_specs=pl.BlockSpec((R, D), lambda i: (i, 0)),
        out_shape=jax.ShapeDtypeStruct((N, D), jnp.float32),
    )(acc, h2, dinv, b2, fcW1, fcb1, fcW2, fcb2)


def kernel(x, edge_index, W1, b1, W2, b2, fcW1, fcb1, fcW2, fcb2):
    src = edge_index[0]
    dst = edge_index[1]
    pad = EPAD - E
    src_p = jnp.concatenate([src, jnp.zeros((pad,), jnp.int32)])
    dst_p = jnp.concatenate([dst, jnp.full((pad,), N, jnp.int32)])
    dst3 = dst_p.reshape(NW, CH, K)
    dstp = dst_p.reshape(NS, 2 * CH, K)
    dst0 = dstp[:, :CH0, :]
    dst1 = dstp[:, CH0:, :]

    p = _deg_kernel(dst3)
    h1, g1, dinv = _tc1(x, W1, p)
    acc1 = _edge_kernel(g1, src_p, dst0, dst1)
    h2, g2 = _tc2(acc1, h1, dinv, b1.reshape(1, D), W2)
    acc2 = _edge_kernel(g2, src_p, dst0, dst1)
    return _tc3(acc2, h2, dinv, b2.reshape(1, D), fcW1, fcb1.reshape(1, D),
                fcW2, fcb2.reshape(1, D))


# speedup vs baseline: 2.2027x; 2.0858x over previous
import jax
import jax.numpy as jnp
from jax import lax
from jax.experimental import pallas as pl
from jax.experimental.pallas import tpu as pltpu
from jax.experimental.pallas import tpu_sc as plsc

N = 10000
D = 128
E = 320000

NC = 2
NS = 16
NW = NC * NS
K = 128
CH = 80
EPW = K * CH
EPAD = EPW * NW
NPAD = 10240
SINK = NPAD - N
ZROWS = NPAD // NS
OROWS = 624
DW = 16
NBUF = 2
LAG = 4

_mesh = plsc.VectorSubcoreMesh(core_axis_name="c", subcore_axis_name="s")


def _deg_body(dst_hbm, out_hbm, dst_all, buf_v, acc_sh, sem0, sem1, sem2,
              sem3, zsem):
    sem = (sem0, sem1, sem2, sem3)
    c = lax.axis_index("c")
    s = lax.axis_index("s")
    wid = s * NC + c

    pltpu.sync_copy(dst_hbm.at[wid], dst_all)

    def _zrow(i, carry):
        for j in range(D // 16):
            buf_v[i, pl.ds(j * 16, 16)] = jnp.zeros((16,), jnp.float32)
        return carry

    lax.fori_loop(0, K, _zrow, 0)
    for m in range(ZROWS // K):
        pltpu.async_copy(buf_v, acc_sh.at[pl.ds(s * ZROWS + m * K, K)], zsem)
    for m in range(ZROWS // K):
        pltpu.make_async_copy(buf_v, acc_sh.at[pl.ds(s * ZROWS + m * K, K)],
                              zsem).wait()

    def _orow(i, carry):
        for j in range(D // 16):
            buf_v[i, pl.ds(j * 16, 16)] = jnp.ones((16,), jnp.float32)
        return carry

    lax.fori_loop(0, K, _orow, 0)
    plsc.subcore_barrier()

    for b in range(LAG):
        pltpu.async_copy(buf_v, acc_sh.at[dst_all.at[b]], sem[b], add=True)

    def _fire(g, carry):
        for b in range(LAG):
            j = g * LAG + b
            pltpu.make_async_copy(buf_v, acc_sh.at[dst_all.at[j - LAG]],
                                  sem[b]).wait()
            pltpu.async_copy(buf_v, acc_sh.at[dst_all.at[j]], sem[b],
                             add=True)
        return carry

    lax.fori_loop(1, CH // LAG, _fire, 0)
    for b in range(LAG):
        pltpu.make_async_copy(buf_v, acc_sh.at[dst_all.at[CH - LAG + b]],
                              sem[b]).wait()
    plsc.subcore_barrier()

    pltpu.sync_copy(acc_sh.at[pl.ds(s * OROWS, OROWS)],
                    out_hbm.at[c, pl.ds(s * OROWS, OROWS)])

    @pl.when(s == NS - 1)
    def _tail():
        pltpu.sync_copy(acc_sh.at[pl.ds(NS * OROWS, N - NS * OROWS)],
                        out_hbm.at[c, pl.ds(NS * OROWS, N - NS * OROWS)])


_deg_kernel = pl.kernel(
    _deg_body,
    out_type=jax.ShapeDtypeStruct((NC, N, D), jnp.float32),
    mesh=_mesh,
    scratch_types=[
        pltpu.VMEM((CH, K), jnp.int32),
        pltpu.VMEM((K, D), jnp.float32),
        pltpu.VMEM_SHARED((NPAD, D), jnp.float32),
    ] + [pltpu.SemaphoreType.DMA] * 5,
)


def _edge_body(g_hbm, src_hbm, dst_hbm, out_hbm, sbuf, dst_all,
               rows_v, acc_sh, isem0, isem1, gsem0, gsem1, ssem0, ssem1,
               zsem):
    isem = (isem0, isem1)
    gsem = (gsem0, gsem1)
    ssem = (ssem0, ssem1)
    c = lax.axis_index("c")
    s = lax.axis_index("s")

    wid = 2 * s + c

    pltpu.sync_copy(dst_hbm.at[wid], dst_all)

    def _zrow(i, carry):
        for j in range(D // 16):
            rows_v[0, i, pl.ds(j * 16, 16)] = jnp.zeros((16,), jnp.float32)
        return carry

    lax.fori_loop(0, K, _zrow, 0)
    for m in range(ZROWS // K):
        pltpu.async_copy(rows_v.at[0], acc_sh.at[pl.ds(s * ZROWS + m * K, K)],
                         zsem)
    for m in range(ZROWS // K):
        pltpu.make_async_copy(rows_v.at[0],
                              acc_sh.at[pl.ds(s * ZROWS + m * K, K)],
                              zsem).wait()
    plsc.subcore_barrier()

    _run_pipeline(g_hbm, src_hbm, sbuf, rows_v, acc_sh, isem, gsem, ssem,
                  dst_all, wid * EPW, CH)

    _edge_out(out_hbm, acc_sh, c, s)


def _run_pipeline(g_hbm, src_hbm, sbuf, rows_v, acc_sh, isem, gsem, ssem,
                  dst_all, ebase, nch):
    for b in range(NBUF):
        pltpu.async_copy(src_hbm.at[pl.ds(ebase + b * K, K)], sbuf.at[b],
                         isem[b])
    for b in range(NBUF):
        pltpu.make_async_copy(src_hbm.at[pl.ds(ebase + b * K, K)], sbuf.at[b],
                              isem[b]).wait()
        pltpu.async_copy(g_hbm.at[sbuf.at[b]], rows_v.at[b], gsem[b])

    def _cycle(g, carry):
        for b in range(NBUF):
            j = g * NBUF + b
            pltpu.make_async_copy(g_hbm.at[sbuf.at[b]], rows_v.at[b],
                                  gsem[b]).wait()
            pltpu.async_copy(rows_v.at[b], acc_sh.at[dst_all.at[j]], ssem[b],
                             add=True)

            @pl.when(j + NBUF < nch)
            def _(b=b, j=j):
                pltpu.async_copy(
                    src_hbm.at[pl.ds(ebase + (j + NBUF) * K, K)], sbuf.at[b],
                    isem[b])

        for b in range(NBUF):
            jn = (g + 1) * NBUF + b

            @pl.when(jn < nch)
            def _(b=b, jn=jn):
                pltpu.make_async_copy(rows_v.at[b],
                                      acc_sh.at[dst_all.at[jn - NBUF]],
                                      ssem[b]).wait()
                pltpu.make_async_copy(
                    src_hbm.at[pl.ds(ebase + jn * K, K)], sbuf.at[b],
                    isem[b]).wait()
                pltpu.async_copy(g_hbm.at[sbuf.at[b]], rows_v.at[b], gsem[b])

        return carry

    lax.fori_loop(0, nch // NBUF, _cycle, 0)
    for b in range(NBUF):
        pltpu.make_async_copy(rows_v.at[b],
                              acc_sh.at[dst_all.at[nch - NBUF + b]],
                              ssem[b]).wait()


def _edge_out(out_hbm, acc_sh, c, s):
    plsc.subcore_barrier()

    pltpu.sync_copy(acc_sh.at[pl.ds(s * OROWS, OROWS)],
                    out_hbm.at[c, pl.ds(s * OROWS, OROWS)])

    @pl.when(s == NS - 1)
    def _tail():
        pltpu.sync_copy(acc_sh.at[pl.ds(NS * OROWS, N - NS * OROWS)],
                        out_hbm.at[c, pl.ds(NS * OROWS, N - NS * OROWS)])


_edge_kernel = pl.kernel(
    _edge_body,
    out_type=jax.ShapeDtypeStruct((NC, N, D), jnp.float32),
    mesh=_mesh,
    scratch_types=[
        pltpu.VMEM((NBUF, K), jnp.int32),
        pltpu.VMEM((CH, K), jnp.int32),
        pltpu.VMEM((NBUF, K, D), jnp.float32),
        pltpu.VMEM_SHARED((NPAD, D), jnp.float32),
    ] + [pltpu.SemaphoreType.DMA] * (3 * NBUF + 1),
)


R = 1000


def _tc1_body(x_ref, w1_ref, p_ref, h_ref, g_ref, dinv_ref):
    p = p_ref[...]
    deg = 1.0 + p[0, :, 0:1] + p[1, :, 0:1]
    dinv = lax.rsqrt(deg)
    h = jnp.dot(x_ref[...], w1_ref[...], preferred_element_type=jnp.float32)
    h_ref[...] = h
    g_ref[...] = h * dinv
    dinv_ref[...] = dinv


def _tc1(x, W1, p):
    return pl.pallas_call(
        _tc1_body,
        grid=(N // R,),
        in_specs=[
            pl.BlockSpec((R, D), lambda i: (i, 0)),
            pl.BlockSpec((D, D), lambda i: (0, 0)),
            pl.BlockSpec((NC, R, D), lambda i: (0, i, 0)),
        ],
        out_specs=[
            pl.BlockSpec((R, D), lambda i: (i, 0)),
            pl.BlockSpec((R, D), lambda i: (i, 0)),
            pl.BlockSpec((R, 1), lambda i: (i, 0)),
        ],
        out_shape=[
            jax.ShapeDtypeStruct((N, D), jnp.float32),
            jax.ShapeDtypeStruct((N, D), jnp.float32),
            jax.ShapeDtypeStruct((N, 1), jnp.float32),
        ],
    )(x, W1, p)


def _tc2_body(acc_ref, h1_ref, dinv_ref, b1_ref, w2_ref, h2_ref, g2_ref):
    dinv = dinv_ref[...]
    a = acc_ref[0] + acc_ref[1]
    z = jnp.maximum(a * dinv + h1_ref[...] * (dinv * dinv) + b1_ref[...], 0.0)
    h2 = jnp.dot(z, w2_ref[...], preferred_element_type=jnp.float32)
    h2_ref[...] = h2
    g2_ref[...] = h2 * dinv


def _tc2(acc, h1, dinv, b1, W2):
    return pl.pallas_call(
        _tc2_body,
        grid=(N // R,),
        in_specs=[
            pl.BlockSpec((NC, R, D), lambda i: (0, i, 0)),
            pl.BlockSpec((R, D), lambda i: (i, 0)),
            pl.BlockSpec((R, 1), lambda i: (i, 0)),
            pl.BlockSpec((1, D), lambda i: (0, 0)),
            pl.BlockSpec((D, D), lambda i: (0, 0)),
        ],
        out_specs=[
            pl.BlockSpec((R, D), lambda i: (i, 0)),
            pl.BlockSpec((R, D), lambda i: (i, 0)),
        ],
        out_shape=[
            jax.ShapeDtypeStruct((N, D), jnp.float32),
            jax.ShapeDtypeStruct((N, D), jnp.float32),
        ],
    )(acc, h1, dinv, b1, W2)


def _tc3_body(acc_ref, h2_ref, dinv_ref, b2_ref, fw1_ref, fb1_ref, fw2_ref,
              fb2_ref, out_ref):
    dinv = dinv_ref[...]
    a = acc_ref[0] + acc_ref[1]
    z = jnp.maximum(a * dinv + h2_ref[...] * (dinv * dinv) + b2_ref[...], 0.0)
    t = jnp.maximum(
        jnp.dot(z, fw1_ref[...], preferred_element_type=jnp.float32)
        + fb1_ref[...], 0.0)
    out_ref[...] = (
        jnp.dot(t, fw2_ref[...], preferred_element_type=jnp.float32)
        + fb2_ref[...])


def _tc3(acc, h2, dinv, b2, fcW1, fcb1, fcW2, fcb2):
    return pl.pallas_call(
        _tc3_body,
        grid=(N // R,),
        in_specs=[
            pl.BlockSpec((NC, R, D), lambda i: (0, i, 0)),
            pl.BlockSpec((R, D), lambda i: (i, 0)),
            pl.BlockSpec((R, 1), lambda i: (i, 0)),
            pl.BlockSpec((1, D), lambda i: (0, 0)),
            pl.BlockSpec((D, D), lambda i: (0, 0)),
            pl.BlockSpec((1, D), lambda i: (0, 0)),
            pl.BlockSpec((D, D), lambda i: (0, 0)),
            pl.BlockSpec((1, D), lambda i: (0, 0)),
        ],
        out_specs=pl.BlockSpec((R, D), lambda i: (i, 0)),
        out_shape=jax.ShapeDtypeStruct((N, D), jnp.float32),
    )(acc, h2, dinv, b2, fcW1, fcb1, fcW2, fcb2)


def kernel(x, edge_index, W1, b1, W2, b2, fcW1, fcb1, fcW2, fcb2):
    src = edge_index[0]
    dst = edge_index[1]
    pad = EPAD - E
    ar = jnp.arange(pad, dtype=jnp.int32)
    src_p = jnp.concatenate([src, ar % N])
    dst_p = jnp.concatenate([dst, N + ar % SINK])
    dst3 = dst_p.reshape(NW, CH, K)

    p = _deg_kernel(dst3)
    h1, g1, dinv = _tc1(x, W1, p)
    acc1 = _edge_kernel(g1, src_p, dst3)
    h2, g2 = _tc2(acc1, h1, dinv, b1.reshape(1, D), W2)
    acc2 = _edge_kernel(g2, src_p, dst3)
    return _tc3(acc2, h2, dinv, b2.reshape(1, D), fcW1, fcb1.reshape(1, D),
                fcW2, fcb2.reshape(1, D))


# speedup vs baseline: 2.6381x; 1.1977x over previous
import jax
import jax.numpy as jnp
from jax import lax
from jax.experimental import pallas as pl
from jax.experimental.pallas import tpu as pltpu
from jax.experimental.pallas import tpu_sc as plsc

N = 10000
D = 128
E = 320000

NC = 2
NS = 16
NW = NC * NS
K = 64
CH = 160
EPW = K * CH
EPAD = EPW * NW
NPAD = 10240
SINK = NPAD - N
ZROWS = NPAD // NS
OROWS = 624
DW = 16
NBUF = 4
LAG = 4

_mesh = plsc.VectorSubcoreMesh(core_axis_name="c", subcore_axis_name="s")


def _deg_body(dst_hbm, out_hbm, dbuf_v, hist_v, tmp_v, res_v, hists_sh):
    c = lax.axis_index("c")
    s = lax.axis_index("s")
    wid = s * NC + c
    seg = NPAD // NS

    pltpu.sync_copy(dst_hbm.at[pl.ds(wid * EPW, EPW)], dbuf_v)

    def _z(i, carry):
        hist_v[pl.ds(i * 16, 16)] = jnp.zeros((16,), jnp.float32)
        return carry

    lax.fori_loop(0, NPAD // 16, _z, 0)

    ones16 = jnp.ones((16,), jnp.float32)

    def _hrow(r, carry):
        for q in range(4):
            idx = dbuf_v[pl.ds((r * 4 + q) * 16, 16)]
            plsc.addupdate_scatter(hist_v, [idx], ones16)
        return carry

    lax.fori_loop(0, EPW // 64, _hrow, 0)

    pltpu.sync_copy(hist_v, hists_sh.at[s])
    plsc.subcore_barrier()
    for t in range(NS):
        pltpu.sync_copy(hists_sh.at[t, pl.ds(s * seg, seg)], tmp_v.at[t])

    def _r(v, carry):
        acc = jnp.zeros((16,), jnp.float32)
        for t in range(NS):
            acc = acc + tmp_v[t, pl.ds(v * 16, 16)]
        res_v[pl.ds(v * 16, 16)] = acc
        return carry

    lax.fori_loop(0, seg // 16, _r, 0)
    pltpu.sync_copy(res_v, out_hbm.at[c, s])


_deg_kernel = pl.kernel(
    _deg_body,
    out_type=jax.ShapeDtypeStruct((NC, NS, NPAD // NS), jnp.float32),
    mesh=_mesh,
    scratch_types=[
        pltpu.VMEM((EPW,), jnp.int32),
        pltpu.VMEM((NPAD,), jnp.float32),
        pltpu.VMEM((NS, NPAD // NS), jnp.float32),
        pltpu.VMEM((NPAD // NS,), jnp.float32),
        pltpu.VMEM_SHARED((NS, NPAD), jnp.float32),
    ],
    compiler_params=pltpu.CompilerParams(needs_layout_passes=False),
)


def _edge_body(g_hbm, src_hbm, dst_hbm, out_hbm, sbuf, dbuf,
               rows_v, acc_sh, isem0, isem1, isem2, isem3, gsem0, gsem1,
               gsem2, gsem3, ssem0, ssem1, ssem2, ssem3, zsem):
    isem = (isem0, isem1, isem2, isem3)
    gsem = (gsem0, gsem1, gsem2, gsem3)
    ssem = (ssem0, ssem1, ssem2, ssem3)
    c = lax.axis_index("c")
    s = lax.axis_index("s")

    wid = 2 * s + c

    def _zrow(i, carry):
        for j in range(D // 16):
            rows_v[0, i, pl.ds(j * 16, 16)] = jnp.zeros((16,), jnp.float32)
        return carry

    lax.fori_loop(0, K, _zrow, 0)
    for m in range(ZROWS // K):
        pltpu.async_copy(rows_v.at[0], acc_sh.at[pl.ds(s * ZROWS + m * K, K)],
                         zsem)
    for m in range(ZROWS // K):
        pltpu.make_async_copy(rows_v.at[0],
                              acc_sh.at[pl.ds(s * ZROWS + m * K, K)],
                              zsem).wait()
    plsc.subcore_barrier()

    _run_pipeline(g_hbm, src_hbm, dst_hbm, sbuf, dbuf, rows_v, acc_sh,
                  isem, gsem, ssem, wid * EPW, CH)

    _edge_out(out_hbm, acc_sh, c, s)


def _run_pipeline(g_hbm, src_hbm, dst_hbm, sbuf, dbuf, rows_v, acc_sh,
                  isem, gsem, ssem, ebase, nch):
    def _idx_start(j, b):
        pltpu.async_copy(src_hbm.at[pl.ds(ebase + j * K, K)], sbuf.at[b],
                         isem[b])
        pltpu.async_copy(dst_hbm.at[pl.ds(ebase + j * K, K)], dbuf.at[b],
                         isem[b])

    def _idx_wait(j, b):
        pltpu.make_async_copy(src_hbm.at[pl.ds(ebase + j * K, K)], sbuf.at[b],
                              isem[b]).wait()
        pltpu.make_async_copy(dst_hbm.at[pl.ds(ebase + j * K, K)], dbuf.at[b],
                              isem[b]).wait()

    for b in range(NBUF):
        _idx_start(b, b)
    for b in range(NBUF):
        _idx_wait(b, b)
        pltpu.async_copy(g_hbm.at[sbuf.at[b]], rows_v.at[b], gsem[b])

    def _cycle(g, carry):
        for b in range(NBUF):
            pltpu.make_async_copy(g_hbm.at[sbuf.at[b]], rows_v.at[b],
                                  gsem[b]).wait()
            pltpu.async_copy(rows_v.at[b], acc_sh.at[dbuf.at[b]], ssem[b],
                             add=True)

        for b in range(NBUF):
            jn = (g + 1) * NBUF + b

            @pl.when(jn < nch)
            def _(b=b, jn=jn):
                pltpu.make_async_copy(rows_v.at[b], acc_sh.at[dbuf.at[b]],
                                      ssem[b]).wait()
                _idx_start(jn, b)
                _idx_wait(jn, b)
                pltpu.async_copy(g_hbm.at[sbuf.at[b]], rows_v.at[b], gsem[b])

        return carry

    lax.fori_loop(0, nch // NBUF, _cycle, 0)
    for b in range(NBUF):
        pltpu.make_async_copy(rows_v.at[b], acc_sh.at[dbuf.at[b]],
                              ssem[b]).wait()


def _edge_out(out_hbm, acc_sh, c, s):
    plsc.subcore_barrier()

    pltpu.sync_copy(acc_sh.at[pl.ds(s * OROWS, OROWS)],
                    out_hbm.at[c, pl.ds(s * OROWS, OROWS)])

    @pl.when(s == NS - 1)
    def _tail():
        pltpu.sync_copy(acc_sh.at[pl.ds(NS * OROWS, N - NS * OROWS)],
                        out_hbm.at[c, pl.ds(NS * OROWS, N - NS * OROWS)])


_edge_kernel = pl.kernel(
    _edge_body,
    out_type=jax.ShapeDtypeStruct((NC, N, D), jnp.float32),
    mesh=_mesh,
    scratch_types=[
        pltpu.VMEM((NBUF, K), jnp.int32),
        pltpu.VMEM((NBUF, K), jnp.int32),
        pltpu.VMEM((NBUF, K, D), jnp.float32),
        pltpu.VMEM_SHARED((NPAD, D), jnp.float32),
    ] + [pltpu.SemaphoreType.DMA] * (3 * NBUF + 1),
)


R = 1000


def _tc1_body(x_ref, w1_ref, p_ref, h_ref, g_ref, dinv_ref):
    p = p_ref[...]
    deg = 1.0 + p[0, :, 0:1] + p[1, :, 0:1]
    dinv = lax.rsqrt(deg)
    h = jnp.dot(x_ref[...], w1_ref[...], preferred_element_type=jnp.float32)
    h_ref[...] = h
    g_ref[...] = h * dinv
    dinv_ref[...] = dinv


def _tc1(x, W1, p):
    return pl.pallas_call(
        _tc1_body,
        grid=(N // R,),
        in_specs=[
            pl.BlockSpec((R, D), lambda i: (i, 0)),
            pl.BlockSpec((D, D), lambda i: (0, 0)),
            pl.BlockSpec((NC, R, 1), lambda i: (0, i, 0)),
        ],
        out_specs=[
            pl.BlockSpec((R, D), lambda i: (i, 0)),
            pl.BlockSpec((R, D), lambda i: (i, 0)),
            pl.BlockSpec((R, 1), lambda i: (i, 0)),
        ],
        out_shape=[
            jax.ShapeDtypeStruct((N, D), jnp.float32),
            jax.ShapeDtypeStruct((N, D), jnp.float32),
            jax.ShapeDtypeStruct((N, 1), jnp.float32),
        ],
    )(x, W1, p)


def _tc2_body(acc_ref, h1_ref, dinv_ref, b1_ref, w2_ref, h2_ref, g2_ref):
    dinv = dinv_ref[...]
    a = acc_ref[0] + acc_ref[1]
    z = jnp.maximum(a * dinv + h1_ref[...] * (dinv * dinv) + b1_ref[...], 0.0)
    h2 = jnp.dot(z, w2_ref[...], preferred_element_type=jnp.float32)
    h2_ref[...] = h2
    g2_ref[...] = h2 * dinv


def _tc2(acc, h1, dinv, b1, W2):
    return pl.pallas_call(
        _tc2_body,
        grid=(N // R,),
        in_specs=[
            pl.BlockSpec((NC, R, D), lambda i: (0, i, 0)),
            pl.BlockSpec((R, D), lambda i: (i, 0)),
            pl.BlockSpec((R, 1), lambda i: (i, 0)),
            pl.BlockSpec((1, D), lambda i: (0, 0)),
            pl.BlockSpec((D, D), lambda i: (0, 0)),
        ],
        out_specs=[
            pl.BlockSpec((R, D), lambda i: (i, 0)),
            pl.BlockSpec((R, D), lambda i: (i, 0)),
        ],
        out_shape=[
            jax.ShapeDtypeStruct((N, D), jnp.float32),
            jax.ShapeDtypeStruct((N, D), jnp.float32),
        ],
    )(acc, h1, dinv, b1, W2)


def _tc3_body(acc_ref, h2_ref, dinv_ref, b2_ref, fw1_ref, fb1_ref, fw2_ref,
              fb2_ref, out_ref):
    dinv = dinv_ref[...]
    a = acc_ref[0] + acc_ref[1]
    z = jnp.maximum(a * dinv + h2_ref[...] * (dinv * dinv) + b2_ref[...], 0.0)
    t = jnp.maximum(
        jnp.dot(z, fw1_ref[...], preferred_element_type=jnp.float32)
        + fb1_ref[...], 0.0)
    out_ref[...] = (
        jnp.dot(t, fw2_ref[...], preferred_element_type=jnp.float32)
        + fb2_ref[...])


def _tc3(acc, h2, dinv, b2, fcW1, fcb1, fcW2, fcb2):
    return pl.pallas_call(
        _tc3_body,
        grid=(N // R,),
        in_specs=[
            pl.BlockSpec((NC, R, D), lambda i: (0, i, 0)),
            pl.BlockSpec((R, D), lambda i: (i, 0)),
            pl.BlockSpec((R, 1), lambda i: (i, 0)),
            pl.BlockSpec((1, D), lambda i: (0, 0)),
            pl.BlockSpec((D, D), lambda i: (0, 0)),
            pl.BlockSpec((1, D), lambda i: (0, 0)),
            pl.BlockSpec((D, D), lambda i: (0, 0)),
            pl.BlockSpec((1, D), lambda i: (0, 0)),
        ],
        out_specs=pl.BlockSpec((R, D), lambda i: (i, 0)),
        out_shape=jax.ShapeDtypeStruct((N, D), jnp.float32),
    )(acc, h2, dinv, b2, fcW1, fcb1, fcW2, fcb2)


def kernel(x, edge_index, W1, b1, W2, b2, fcW1, fcb1, fcW2, fcb2):
    src = edge_index[0]
    dst = edge_index[1]
    pad = EPAD - E
    ar = jnp.arange(pad, dtype=jnp.int32)
    src_p = jnp.concatenate([src, ar % N])
    dst_p = jnp.concatenate([dst, N + ar % SINK])
    dst3 = dst_p.reshape(NW, CH, K)

    p = _deg_kernel(dst_p)
    p = p.reshape(NC, NPAD)[:, :N].reshape(NC, N, 1)
    h1, g1, dinv = _tc1(x, W1, p)
    acc1 = _edge_kernel(g1, src_p, dst_p)
    h2, g2 = _tc2(acc1, h1, dinv, b1.reshape(1, D), W2)
    acc2 = _edge_kernel(g2, src_p, dst_p)
    return _tc3(acc2, h2, dinv, b2.reshape(1, D), fcW1, fcb1.reshape(1, D),
                fcW2, fcb2.reshape(1, D))
